# baseline (speedup 1.0000x reference)
import functools

import jax
import jax.numpy as jnp
from jax import lax
from jax.experimental import pallas as pl
from jax.experimental.pallas import tpu as pltpu
from jax.experimental.pallas import tpu_sc as plsc

BATCH = 4096
DIM = 1024

_INFO = plsc.get_sparse_core_info()
_NC = _INFO.num_cores
_NS = _INFO.num_subcores
_NW = _NC * _NS
_BPW = BATCH // _NW
_CB = 8
_NCHUNK = _BPW // _CB
_NBUF = 8
_GAHEAD = 4

_MESH = plsc.VectorSubcoreMesh(core_axis_name="c", subcore_axis_name="s")


@functools.partial(
    pl.kernel,
    mesh=_MESH,
    out_type=jax.ShapeDtypeStruct((3, BATCH, DIM), jnp.float32),
    scratch_types=(
        [pltpu.VMEM((3, _BPW), jnp.int32)]
        + [pltpu.VMEM((_CB, DIM), jnp.float32)] * _NBUF
        + [pltpu.SemaphoreType.DMA] * (2 * _NBUF)
    ),
)
def _sc_prefix_gather(ids_hbm, ent_hbm, rel_hbm, out_hbm, idx_v, *rest):
    wid = lax.axis_index("s") * _NC + lax.axis_index("c")
    base = wid * _BPW

    pltpu.sync_copy(ids_hbm.at[:, pl.ds(base, _BPW)], idx_v)

    bufs = rest[:_NBUF]
    gsems = rest[_NBUF:2 * _NBUF]
    osems = rest[2 * _NBUF:]
    nbuf = _NBUF
    units = [(j, c) for j in range(3) for c in range(_NCHUNK)]
    n = len(units)

    def start_gather(i):
        j, c = units[i]
        table = rel_hbm if j == 1 else ent_hbm
        return pltpu.async_copy(
            table.at[idx_v.at[j, pl.ds(c * _CB, _CB)]],
            bufs[i % nbuf], gsems[i % nbuf])

    def start_put(i):
        j, c = units[i]
        return pltpu.async_copy(
            bufs[i % nbuf],
            out_hbm.at[j, pl.ds(base + c * _CB, _CB)],
            osems[i % nbuf])

    gathers = [None] * n
    puts = [None] * n
    put_waited = [False] * n
    for i in range(min(_GAHEAD, n)):
        gathers[i] = start_gather(i)
    for i in range(n):
        k = i + _GAHEAD
        if k < n:
            if k - nbuf >= 0:
                puts[k - nbuf].wait()
                put_waited[k - nbuf] = True
            gathers[k] = start_gather(k)
        gathers[i].wait()
        puts[i] = start_put(i)
    for i in range(n):
        if not put_waited[i]:
            puts[i].wait()


def kernel(triple_ids, ent_embeddings, rel_embeddings):
    ids_t = triple_ids.T
    out = _sc_prefix_gather(ids_t, ent_embeddings, rel_embeddings)
    return jnp.transpose(out, (1, 0, 2))


# speedup vs baseline: 1.0321x; 1.0321x over previous
import functools

import jax
import jax.numpy as jnp
from jax import lax
from jax.experimental import pallas as pl
from jax.experimental.pallas import tpu as pltpu
from jax.experimental.pallas import tpu_sc as plsc

BATCH = 4096
DIM = 1024

_INFO = plsc.get_sparse_core_info()
_NC = _INFO.num_cores
_NS = _INFO.num_subcores
_NW = _NC * _NS
_BPW = BATCH // _NW
_CB = 16
_NCHUNK = _BPW // _CB
_NBUF = 6
_GAHEAD = 5

_MESH = plsc.VectorSubcoreMesh(core_axis_name="c", subcore_axis_name="s")


@functools.partial(
    pl.kernel,
    mesh=_MESH,
    out_type=jax.ShapeDtypeStruct((3, BATCH, DIM), jnp.float32),
    scratch_types=(
        [pltpu.VMEM((3, _BPW), jnp.int32)]
        + [pltpu.VMEM((_CB, DIM), jnp.float32)] * _NBUF
        + [pltpu.SemaphoreType.DMA] * (2 * _NBUF)
    ),
)
def _sc_prefix_gather(ids_hbm, ent_hbm, rel_hbm, out_hbm, idx_v, *rest):
    wid = lax.axis_index("s") * _NC + lax.axis_index("c")
    base = wid * _BPW

    pltpu.sync_copy(ids_hbm.at[:, pl.ds(base, _BPW)], idx_v)

    bufs = rest[:_NBUF]
    gsems = rest[_NBUF:2 * _NBUF]
    osems = rest[2 * _NBUF:]
    nbuf = _NBUF
    units = [(j, c) for j in range(3) for c in range(_NCHUNK)]
    n = len(units)

    def start_gather(i):
        j, c = units[i]
        table = rel_hbm if j == 1 else ent_hbm
        return pltpu.async_copy(
            table.at[idx_v.at[j, pl.ds(c * _CB, _CB)]],
            bufs[i % nbuf], gsems[i % nbuf])

    def start_put(i):
        j, c = units[i]
        return pltpu.async_copy(
            bufs[i % nbuf],
            out_hbm.at[j, pl.ds(base + c * _CB, _CB)],
            osems[i % nbuf])

    gathers = [None] * n
    puts = [None] * n
    put_waited = [False] * n
    for i in range(min(_GAHEAD, n)):
        gathers[i] = start_gather(i)
    for i in range(n):
        k = i + _GAHEAD
        if k < n:
            if k - nbuf >= 0:
                puts[k - nbuf].wait()
                put_waited[k - nbuf] = True
            gathers[k] = start_gather(k)
        gathers[i].wait()
        puts[i] = start_put(i)
    for i in range(n):
        if not put_waited[i]:
            puts[i].wait()


def kernel(triple_ids, ent_embeddings, rel_embeddings):
    ids_t = triple_ids.T
    out = _sc_prefix_gather(ids_t, ent_embeddings, rel_embeddings)
    return jnp.transpose(out, (1, 0, 2))
